# baseline (speedup 1.0000x reference)
import jax
import jax.numpy as jnp
from jax import lax
from jax.experimental import pallas as pl
from jax.experimental.pallas import tpu as pltpu
from jax.experimental.pallas import tpu_sc as plsc

E = 6_400_000
N = 100_000
W = 3 * N
NC = 2
NS = 16
L = 16
NW = NC * NS
CHUNK = 2048
CW = CHUNK * 3
GROUPS = CHUNK // L
TOTAL_CHUNKS = E // CHUNK
BASE_CHUNKS = TOTAL_CHUNKS // NW
EXTRA = TOTAL_CHUNKS % NW
WPS = 18752
WPS_LAST = W - (NS - 1) * WPS


def _sc_body(diff_hbm, idx_hbm, zeros_hbm, out_hbm,
             diff_v, idx_v, pos_v, neg_v, iiw_v, jjw_v, acc_s, sem_sc):
    cid = lax.axis_index("c")
    sid = lax.axis_index("s")
    wid = cid * NS + sid

    r0 = sid * WPS

    @pl.when(sid < NS - 1)
    def _():
        pltpu.sync_copy(zeros_hbm.at[pl.ds(r0, WPS)], acc_s.at[pl.ds(r0, WPS)])

    @pl.when(sid == NS - 1)
    def _():
        pltpu.sync_copy(zeros_hbm.at[pl.ds((NS - 1) * WPS, WPS_LAST)],
                        acc_s.at[pl.ds((NS - 1) * WPS, WPS_LAST)])

    plsc.subcore_barrier()

    start = wid * BASE_CHUNKS + jnp.minimum(wid, EXTRA)
    nchunks = BASE_CHUNKS + (wid < EXTRA).astype(jnp.int32)

    iota = lax.iota(jnp.int32, L)
    magic = jnp.full((L,), 0x5F3759DF, jnp.int32)

    c0 = jnp.zeros((L,), jnp.int32)
    c1 = jnp.full((L,), 1, jnp.int32)
    c2 = jnp.full((L,), 2, jnp.int32)

    def do_chunk(ci, carry):
        e0 = (start + ci) * CHUNK
        pltpu.sync_copy(diff_hbm.at[pl.ds(e0, CHUNK), :], diff_v)
        pltpu.sync_copy(idx_hbm.at[pl.ds(e0, CHUNK), :], idx_v)

        def do_group(g, c_):
            rows = g * L + iota
            p0 = rows * 3
            p1 = p0 + 1
            p2 = p0 + 2
            dx = plsc.load_gather(diff_v, [rows, c0])
            dy = plsc.load_gather(diff_v, [rows, c1])
            dz = plsc.load_gather(diff_v, [rows, c2])
            ii = plsc.load_gather(idx_v, [rows, c0])
            jj = plsc.load_gather(idx_v, [rows, c1])
            r2 = dx * dx + dy * dy + dz * dz + 1e-12
            bi = plsc.bitcast(r2, jnp.int32)
            y = plsc.bitcast(magic - lax.shift_right_logical(bi, 1), jnp.float32)
            xh = r2 * 0.5
            y = y * (1.5 - xh * y * y)
            y = y * (1.5 - xh * y * y)
            y = y * (1.5 - xh * y * y)
            s = 1.0 - y
            t = y - 1.0
            plsc.store_scatter(pos_v, [p0], s * dx)
            plsc.store_scatter(pos_v, [p1], s * dy)
            plsc.store_scatter(pos_v, [p2], s * dz)
            plsc.store_scatter(neg_v, [p0], t * dx)
            plsc.store_scatter(neg_v, [p1], t * dy)
            plsc.store_scatter(neg_v, [p2], t * dz)
            wa = ii * 3
            wb = jj * 3
            plsc.store_scatter(iiw_v, [p0], wa)
            plsc.store_scatter(iiw_v, [p1], wa + 1)
            plsc.store_scatter(iiw_v, [p2], wa + 2)
            plsc.store_scatter(jjw_v, [p0], wb)
            plsc.store_scatter(jjw_v, [p1], wb + 1)
            plsc.store_scatter(jjw_v, [p2], wb + 2)
            return c_

        lax.fori_loop(0, GROUPS, do_group, 0, unroll=False)

        dpos = pltpu.async_copy(pos_v, acc_s.at[iiw_v], sem_sc, add=True)
        dneg = pltpu.async_copy(neg_v, acc_s.at[jjw_v], sem_sc, add=True)
        dpos.wait()
        dneg.wait()
        return carry

    lax.fori_loop(0, nchunks, do_chunk, 0, unroll=False)

    plsc.subcore_barrier()

    @pl.when(sid < NS - 1)
    def _():
        pltpu.sync_copy(acc_s.at[pl.ds(r0, WPS)],
                        out_hbm.at[cid, pl.ds(r0, WPS)])

    @pl.when(sid == NS - 1)
    def _():
        pltpu.sync_copy(acc_s.at[pl.ds((NS - 1) * WPS, WPS_LAST)],
                        out_hbm.at[cid, pl.ds((NS - 1) * WPS, WPS_LAST)])


def _combine_body(a_ref, b_ref, o_ref):
    o_ref[...] = a_ref[...] + b_ref[...]


def kernel(edge_diff, edge_idx, n_atoms):
    del n_atoms
    zeros = jnp.zeros((W,), jnp.float32)
    mesh = plsc.VectorSubcoreMesh(core_axis_name="c", subcore_axis_name="s")
    partials = pl.kernel(
        _sc_body,
        out_type=jax.ShapeDtypeStruct((NC, W), jnp.float32),
        compiler_params=pltpu.CompilerParams(
            needs_layout_passes=False, use_tc_tiling_on_sc=False),
        mesh=mesh,
        scratch_types=[
            pltpu.VMEM((CHUNK, 3), jnp.float32),
            pltpu.VMEM((CHUNK, 2), jnp.int32),
            pltpu.VMEM((CW,), jnp.float32),
            pltpu.VMEM((CW,), jnp.float32),
            pltpu.VMEM((CW,), jnp.int32),
            pltpu.VMEM((CW,), jnp.int32),
            pltpu.VMEM_SHARED((W,), jnp.float32),
            pltpu.SemaphoreType.DMA,
        ],
    )(edge_diff, edge_idx, zeros)

    pa = partials[0].reshape(300, 1000)
    pb = partials[1].reshape(300, 1000)
    out = pl.pallas_call(
        _combine_body,
        out_shape=jax.ShapeDtypeStruct((300, 1000), jnp.float32),
    )(pa, pb)
    return out.reshape(N, 3)


# speedup vs baseline: 18.8715x; 18.8715x over previous
import jax
import jax.numpy as jnp
from jax import lax
from jax.experimental import pallas as pl
from jax.experimental.pallas import tpu as pltpu
from jax.experimental.pallas import tpu_sc as plsc

E = 6_400_000
N = 100_000
W = 3 * N
NC = 2
NS = 16
L = 16
NW = NC * NS
CHUNK = 2048
GROUPS = CHUNK // L
TOTAL_CHUNKS = E // CHUNK
BASE_CHUNKS = TOTAL_CHUNKS // NW
EXTRA = TOTAL_CHUNKS % NW
WPS = 18752
WPS_LAST = W - (NS - 1) * WPS


def _sc_body(dx_hbm, dy_hbm, dz_hbm, ii_hbm, jj_hbm, zeros_hbm, out_hbm,
             vx, vy, vz, vii, vjj,
             px, py, pz, nx, ny, nz,
             wi0, wi1, wi2, wj0, wj1, wj2,
             acc_s, sem_sc):
    cid = lax.axis_index("c")
    sid = lax.axis_index("s")
    wid = cid * NS + sid

    r0 = sid * WPS

    @pl.when(sid < NS - 1)
    def _():
        pltpu.sync_copy(zeros_hbm.at[pl.ds(r0, WPS)], acc_s.at[pl.ds(r0, WPS)])

    @pl.when(sid == NS - 1)
    def _():
        pltpu.sync_copy(zeros_hbm.at[pl.ds((NS - 1) * WPS, WPS_LAST)],
                        acc_s.at[pl.ds((NS - 1) * WPS, WPS_LAST)])

    plsc.subcore_barrier()

    start = wid * BASE_CHUNKS + jnp.minimum(wid, EXTRA)
    nchunks = BASE_CHUNKS + (wid < EXTRA).astype(jnp.int32)

    magic = jnp.full((L,), 0x5F3759DF, jnp.int32)

    def do_chunk(ci, carry):
        e0 = (start + ci) * CHUNK
        pltpu.sync_copy(dx_hbm.at[pl.ds(e0, CHUNK)], vx)
        pltpu.sync_copy(dy_hbm.at[pl.ds(e0, CHUNK)], vy)
        pltpu.sync_copy(dz_hbm.at[pl.ds(e0, CHUNK)], vz)
        pltpu.sync_copy(ii_hbm.at[pl.ds(e0, CHUNK)], vii)
        pltpu.sync_copy(jj_hbm.at[pl.ds(e0, CHUNK)], vjj)

        def do_group(g, c_):
            sl = pl.ds(g * L, L)
            ax = vx[sl]
            ay = vy[sl]
            az = vz[sl]
            r2 = ax * ax + ay * ay + az * az + 1e-12
            bi = plsc.bitcast(r2, jnp.int32)
            y = plsc.bitcast(magic - lax.shift_right_logical(bi, 1), jnp.float32)
            xh = r2 * 0.5
            y = y * (1.5 - xh * y * y)
            y = y * (1.5 - xh * y * y)
            y = y * (1.5 - xh * y * y)
            s = 1.0 - y
            t = y - 1.0
            px[sl] = s * ax
            py[sl] = s * ay
            pz[sl] = s * az
            nx[sl] = t * ax
            ny[sl] = t * ay
            nz[sl] = t * az
            wa = vii[sl] * 3
            wb = vjj[sl] * 3
            wi0[sl] = wa
            wi1[sl] = wa + 1
            wi2[sl] = wa + 2
            wj0[sl] = wb
            wj1[sl] = wb + 1
            wj2[sl] = wb + 2
            return c_

        lax.fori_loop(0, GROUPS, do_group, 0, unroll=False)

        descs = [
            pltpu.async_copy(px, acc_s.at[wi0], sem_sc, add=True),
            pltpu.async_copy(py, acc_s.at[wi1], sem_sc, add=True),
            pltpu.async_copy(pz, acc_s.at[wi2], sem_sc, add=True),
            pltpu.async_copy(nx, acc_s.at[wj0], sem_sc, add=True),
            pltpu.async_copy(ny, acc_s.at[wj1], sem_sc, add=True),
            pltpu.async_copy(nz, acc_s.at[wj2], sem_sc, add=True),
        ]
        for d_ in descs:
            d_.wait()
        return carry

    lax.fori_loop(0, nchunks, do_chunk, 0, unroll=False)

    plsc.subcore_barrier()

    @pl.when(sid < NS - 1)
    def _():
        pltpu.sync_copy(acc_s.at[pl.ds(r0, WPS)],
                        out_hbm.at[cid, pl.ds(r0, WPS)])

    @pl.when(sid == NS - 1)
    def _():
        pltpu.sync_copy(acc_s.at[pl.ds((NS - 1) * WPS, WPS_LAST)],
                        out_hbm.at[cid, pl.ds((NS - 1) * WPS, WPS_LAST)])


def _combine_body(a_ref, b_ref, o_ref):
    o_ref[...] = a_ref[...] + b_ref[...]


def kernel(edge_diff, edge_idx, n_atoms):
    del n_atoms
    dx = edge_diff[:, 0]
    dy = edge_diff[:, 1]
    dz = edge_diff[:, 2]
    ii = edge_idx[:, 0]
    jj = edge_idx[:, 1]
    zeros = jnp.zeros((W,), jnp.float32)
    mesh = plsc.VectorSubcoreMesh(core_axis_name="c", subcore_axis_name="s")
    fvec = pltpu.VMEM((CHUNK,), jnp.float32)
    ivec = pltpu.VMEM((CHUNK,), jnp.int32)
    partials = pl.kernel(
        _sc_body,
        out_type=jax.ShapeDtypeStruct((NC, W), jnp.float32),
        compiler_params=pltpu.CompilerParams(
            needs_layout_passes=False, use_tc_tiling_on_sc=False),
        mesh=mesh,
        scratch_types=[
            fvec, fvec, fvec, ivec, ivec,
            fvec, fvec, fvec, fvec, fvec, fvec,
            ivec, ivec, ivec, ivec, ivec, ivec,
            pltpu.VMEM_SHARED((W,), jnp.float32),
            pltpu.SemaphoreType.DMA,
        ],
    )(dx, dy, dz, ii, jj, zeros)

    pa = partials[0].reshape(300, 1000)
    pb = partials[1].reshape(300, 1000)
    out = pl.pallas_call(
        _combine_body,
        out_shape=jax.ShapeDtypeStruct((300, 1000), jnp.float32),
    )(pa, pb)
    return out.reshape(N, 3)


# speedup vs baseline: 31.3110x; 1.6592x over previous
import jax
import jax.numpy as jnp
from jax import lax
from jax.experimental import pallas as pl
from jax.experimental.pallas import tpu as pltpu
from jax.experimental.pallas import tpu_sc as plsc

E = 6_400_000
N = 100_000
W = 3 * N
NC = 2
NS = 16
L = 16
NW = NC * NS
CHUNK = 2000
GROUPS = CHUNK // L
NCH = E // CHUNK // NW
WPS = 18752
WPS_LAST = W - (NS - 1) * WPS


def _sc_body(dx_hbm, dy_hbm, dz_hbm, ii_hbm, jj_hbm, zeros_hbm, out_hbm,
             *scr):
    ins = (scr[0:5], scr[5:10])
    vals = (scr[10:16], scr[16:22])
    idxs = (scr[22:28], scr[28:34])
    acc_s = scr[34]
    sem_in = (scr[35], scr[36])
    sem_st = (scr[37], scr[38])

    cid = lax.axis_index("c")
    sid = lax.axis_index("s")
    wid = cid * NS + sid

    r0 = sid * WPS

    @pl.when(sid < NS - 1)
    def _():
        pltpu.sync_copy(zeros_hbm.at[pl.ds(r0, WPS)], acc_s.at[pl.ds(r0, WPS)])

    @pl.when(sid == NS - 1)
    def _():
        pltpu.sync_copy(zeros_hbm.at[pl.ds((NS - 1) * WPS, WPS_LAST)],
                        acc_s.at[pl.ds((NS - 1) * WPS, WPS_LAST)])

    plsc.subcore_barrier()

    start = wid * NCH
    magic = jnp.full((L,), 0x5F3759DF, jnp.int32)
    srcs = (dx_hbm, dy_hbm, dz_hbm, ii_hbm, jj_hbm)

    def fire_inputs(ci, k):
        e0 = (start + ci) * CHUNK
        for src, dst in zip(srcs, ins[k]):
            pltpu.async_copy(src.at[pl.ds(e0, CHUNK)], dst, sem_in[k])

    def wait_inputs(k):
        for src, dst in zip(srcs, ins[k]):
            pltpu.make_async_copy(src.at[pl.ds(0, CHUNK)], dst, sem_in[k]).wait()

    def fire_streams(k):
        px, py, pz, nx, ny, nz = vals[k]
        wi0, wi1, wi2, wj0, wj1, wj2 = idxs[k]
        for v, w in ((px, wi0), (py, wi1), (pz, wi2),
                     (nx, wj0), (ny, wj1), (nz, wj2)):
            pltpu.async_copy(v, acc_s.at[w], sem_st[k], add=True)

    def wait_streams(k):
        px, py, pz, nx, ny, nz = vals[k]
        wi0, wi1, wi2, wj0, wj1, wj2 = idxs[k]
        for v, w in ((px, wi0), (py, wi1), (pz, wi2),
                     (nx, wj0), (ny, wj1), (nz, wj2)):
            pltpu.make_async_copy(v, acc_s.at[w], sem_st[k]).wait()

    def compute(k):
        vx, vy, vz, vii, vjj = ins[k]
        px, py, pz, nx, ny, nz = vals[k]
        wi0, wi1, wi2, wj0, wj1, wj2 = idxs[k]

        def do_group(g, c_):
            sl = pl.ds(g * L, L)
            ax = vx[sl]
            ay = vy[sl]
            az = vz[sl]
            r2 = ax * ax + ay * ay + az * az + 1e-12
            bi = plsc.bitcast(r2, jnp.int32)
            y = plsc.bitcast(magic - lax.shift_right_logical(bi, 1), jnp.float32)
            xh = r2 * 0.5
            y = y * (1.5 - xh * y * y)
            y = y * (1.5 - xh * y * y)
            y = y * (1.5 - xh * y * y)
            s = 1.0 - y
            t = y - 1.0
            px[sl] = s * ax
            py[sl] = s * ay
            pz[sl] = s * az
            nx[sl] = t * ax
            ny[sl] = t * ay
            nz[sl] = t * az
            wa = vii[sl] * 3
            wb = vjj[sl] * 3
            wi0[sl] = wa
            wi1[sl] = wa + 1
            wi2[sl] = wa + 2
            wj0[sl] = wb
            wj1[sl] = wb + 1
            wj2[sl] = wb + 2
            return c_

        lax.fori_loop(0, GROUPS, do_group, 0, unroll=False)

    fire_inputs(0, 0)

    def do_chunk(ci, carry):
        def phase(k):
            wait_inputs(k)

            @pl.when(ci < NCH - 1)
            def _():
                fire_inputs(ci + 1, 1 - k)

            @pl.when(ci >= 2)
            def _():
                wait_streams(k)

            compute(k)
            fire_streams(k)

        @pl.when(ci % 2 == 0)
        def _():
            phase(0)

        @pl.when(ci % 2 == 1)
        def _():
            phase(1)

        return carry

    lax.fori_loop(0, NCH, do_chunk, 0, unroll=False)
    wait_streams(0)
    wait_streams(1)

    plsc.subcore_barrier()

    @pl.when(sid < NS - 1)
    def _():
        pltpu.sync_copy(acc_s.at[pl.ds(r0, WPS)],
                        out_hbm.at[cid, pl.ds(r0, WPS)])

    @pl.when(sid == NS - 1)
    def _():
        pltpu.sync_copy(acc_s.at[pl.ds((NS - 1) * WPS, WPS_LAST)],
                        out_hbm.at[cid, pl.ds((NS - 1) * WPS, WPS_LAST)])


def _combine_body(a_ref, b_ref, o_ref):
    o_ref[...] = a_ref[...] + b_ref[...]


def kernel(edge_diff, edge_idx, n_atoms):
    del n_atoms
    dx = edge_diff[:, 0]
    dy = edge_diff[:, 1]
    dz = edge_diff[:, 2]
    ii = edge_idx[:, 0]
    jj = edge_idx[:, 1]
    zeros = jnp.zeros((W,), jnp.float32)
    mesh = plsc.VectorSubcoreMesh(core_axis_name="c", subcore_axis_name="s")
    fvec = pltpu.VMEM((CHUNK,), jnp.float32)
    ivec = pltpu.VMEM((CHUNK,), jnp.int32)
    in_set = [fvec, fvec, fvec, ivec, ivec]
    val_set = [fvec] * 6
    idx_set = [ivec] * 6
    partials = pl.kernel(
        _sc_body,
        out_type=jax.ShapeDtypeStruct((NC, W), jnp.float32),
        compiler_params=pltpu.CompilerParams(
            needs_layout_passes=False, use_tc_tiling_on_sc=False),
        mesh=mesh,
        scratch_types=(
            in_set + in_set + val_set + val_set + idx_set + idx_set
            + [pltpu.VMEM_SHARED((W,), jnp.float32)]
            + [pltpu.SemaphoreType.DMA] * 4
        ),
    )(dx, dy, dz, ii, jj, zeros)

    pa = partials[0].reshape(300, 1000)
    pb = partials[1].reshape(300, 1000)
    out = pl.pallas_call(
        _combine_body,
        out_shape=jax.ShapeDtypeStruct((300, 1000), jnp.float32),
    )(pa, pb)
    return out.reshape(N, 3)


# speedup vs baseline: 31.3164x; 1.0002x over previous
import jax
import jax.numpy as jnp
from jax import lax
from jax.experimental import pallas as pl
from jax.experimental.pallas import tpu as pltpu
from jax.experimental.pallas import tpu_sc as plsc

E = 6_400_000
N = 100_000
W = 3 * N
NC = 2
NS = 16
L = 16
NW = NC * NS
CHUNK = 2000
GROUPS = CHUNK // L
NCH = E // CHUNK // NW
WPS = 18752
WPS_LAST = W - (NS - 1) * WPS


def _sc_body(dx_hbm, dy_hbm, dz_hbm, ii_hbm, jj_hbm, zeros_hbm, out_hbm,
             *scr):
    ins = (scr[0:5], scr[5:10])
    vals = (scr[10:12], scr[12:14])
    idxs = (scr[14:16], scr[16:18])
    acc_s = scr[18]
    sem_in = (scr[19], scr[20])
    sem_st = (scr[21], scr[22])

    cid = lax.axis_index("c")
    sid = lax.axis_index("s")
    wid = cid * NS + sid

    r0 = sid * WPS

    @pl.when(sid < NS - 1)
    def _():
        pltpu.sync_copy(zeros_hbm.at[pl.ds(r0, WPS)], acc_s.at[pl.ds(r0, WPS)])

    @pl.when(sid == NS - 1)
    def _():
        pltpu.sync_copy(zeros_hbm.at[pl.ds((NS - 1) * WPS, WPS_LAST)],
                        acc_s.at[pl.ds((NS - 1) * WPS, WPS_LAST)])

    plsc.subcore_barrier()

    start = wid * NCH
    magic = jnp.full((L,), 0x5F3759DF, jnp.int32)
    srcs = (dx_hbm, dy_hbm, dz_hbm, ii_hbm, jj_hbm)

    def fire_inputs(ci, k):
        e0 = (start + ci) * CHUNK
        for src, dst in zip(srcs, ins[k]):
            pltpu.async_copy(src.at[pl.ds(e0, CHUNK)], dst, sem_in[k])

    def wait_inputs(k):
        for src, dst in zip(srcs, ins[k]):
            pltpu.make_async_copy(src.at[pl.ds(0, CHUNK)], dst, sem_in[k]).wait()

    def fire_streams(k):
        pbuf, nbuf = vals[k]
        wibuf, wjbuf = idxs[k]
        pltpu.async_copy(pbuf, acc_s.at[wibuf], sem_st[k], add=True)
        pltpu.async_copy(nbuf, acc_s.at[wjbuf], sem_st[k], add=True)

    def wait_streams(k):
        pbuf, nbuf = vals[k]
        wibuf, wjbuf = idxs[k]
        pltpu.make_async_copy(pbuf, acc_s.at[wibuf], sem_st[k]).wait()
        pltpu.make_async_copy(nbuf, acc_s.at[wjbuf], sem_st[k]).wait()

    def compute(k):
        vx, vy, vz, vii, vjj = ins[k]
        pbuf, nbuf = vals[k]
        wibuf, wjbuf = idxs[k]

        def do_group(g, c_):
            o = g * L
            sl = pl.ds(o, L)
            sx = pl.ds(o, L)
            sy = pl.ds(o + CHUNK, L)
            sz = pl.ds(o + 2 * CHUNK, L)
            ax = vx[sl]
            ay = vy[sl]
            az = vz[sl]
            r2 = ax * ax + ay * ay + az * az + 1e-12
            bi = plsc.bitcast(r2, jnp.int32)
            y = plsc.bitcast(magic - lax.shift_right_logical(bi, 1), jnp.float32)
            xh = r2 * 0.5
            y = y * (1.5 - xh * y * y)
            y = y * (1.5 - xh * y * y)
            y = y * (1.5 - xh * y * y)
            s = 1.0 - y
            t = y - 1.0
            pbuf[sx] = s * ax
            pbuf[sy] = s * ay
            pbuf[sz] = s * az
            nbuf[sx] = t * ax
            nbuf[sy] = t * ay
            nbuf[sz] = t * az
            wa = vii[sl] * 3
            wb = vjj[sl] * 3
            wibuf[sx] = wa
            wibuf[sy] = wa + 1
            wibuf[sz] = wa + 2
            wjbuf[sx] = wb
            wjbuf[sy] = wb + 1
            wjbuf[sz] = wb + 2
            return c_

        lax.fori_loop(0, GROUPS, do_group, 0, unroll=False)

    fire_inputs(0, 0)

    def do_chunk(ci, carry):
        def phase(k):
            wait_inputs(k)

            @pl.when(ci < NCH - 1)
            def _():
                fire_inputs(ci + 1, 1 - k)

            @pl.when(ci >= 2)
            def _():
                wait_streams(k)

            compute(k)
            fire_streams(k)

        @pl.when(ci % 2 == 0)
        def _():
            phase(0)

        @pl.when(ci % 2 == 1)
        def _():
            phase(1)

        return carry

    lax.fori_loop(0, NCH, do_chunk, 0, unroll=False)
    wait_streams(0)
    wait_streams(1)

    plsc.subcore_barrier()

    @pl.when(sid < NS - 1)
    def _():
        pltpu.sync_copy(acc_s.at[pl.ds(r0, WPS)],
                        out_hbm.at[cid, pl.ds(r0, WPS)])

    @pl.when(sid == NS - 1)
    def _():
        pltpu.sync_copy(acc_s.at[pl.ds((NS - 1) * WPS, WPS_LAST)],
                        out_hbm.at[cid, pl.ds((NS - 1) * WPS, WPS_LAST)])


def _combine_body(a_ref, b_ref, o_ref):
    o_ref[...] = a_ref[...] + b_ref[...]


def kernel(edge_diff, edge_idx, n_atoms):
    del n_atoms
    dx = edge_diff[:, 0]
    dy = edge_diff[:, 1]
    dz = edge_diff[:, 2]
    ii = edge_idx[:, 0]
    jj = edge_idx[:, 1]
    zeros = jnp.zeros((W,), jnp.float32)
    mesh = plsc.VectorSubcoreMesh(core_axis_name="c", subcore_axis_name="s")
    fvec = pltpu.VMEM((CHUNK,), jnp.float32)
    ivec = pltpu.VMEM((CHUNK,), jnp.int32)
    f3vec = pltpu.VMEM((3 * CHUNK,), jnp.float32)
    i3vec = pltpu.VMEM((3 * CHUNK,), jnp.int32)
    in_set = [fvec, fvec, fvec, ivec, ivec]
    partials = pl.kernel(
        _sc_body,
        out_type=jax.ShapeDtypeStruct((NC, W), jnp.float32),
        compiler_params=pltpu.CompilerParams(
            needs_layout_passes=False, use_tc_tiling_on_sc=False),
        mesh=mesh,
        scratch_types=(
            in_set + in_set
            + [f3vec, f3vec] + [f3vec, f3vec]
            + [i3vec, i3vec] + [i3vec, i3vec]
            + [pltpu.VMEM_SHARED((W,), jnp.float32)]
            + [pltpu.SemaphoreType.DMA] * 4
        ),
    )(dx, dy, dz, ii, jj, zeros)

    pa = partials[0].reshape(300, 1000)
    pb = partials[1].reshape(300, 1000)
    out = pl.pallas_call(
        _combine_body,
        out_shape=jax.ShapeDtypeStruct((300, 1000), jnp.float32),
    )(pa, pb)
    return out.reshape(N, 3)
